# Initial kernel scaffold; baseline (speedup 1.0000x reference)
import jax
import jax.numpy as jnp
from jax.experimental import pallas as pl


def kernel(batch1, batch2, features, adj, mlp_W0, mlp_b0, neigh_W0, self_W0, mlp_W1, mlp_b1, neigh_W1, self_W1):
    raise NotImplementedError("write your pallas kernel here")


# speedup vs baseline: 2.9606x; 2.9606x over previous
import functools

import jax
import jax.numpy as jnp
from jax import lax
from jax.experimental import pallas as pl
from jax.experimental.pallas import tpu as pltpu
from jax.experimental.pallas import tpu_sc as plsc

N_NODES = 10000
MAX_DEG = 32
D_FEAT = 128
HIDDEN = 512
S_HOP2 = 25
S_HOP1 = 10
BATCH = 512
NW = 32


def _make_sc_gather(V, D, B, dtype, chunk):
    per_w = B // NW
    assert B % NW == 0 and per_w % chunk == 0 and chunk % 8 == 0 and chunk <= 128
    n_chunks = per_w // chunk
    mesh = plsc.VectorSubcoreMesh(core_axis_name="c", subcore_axis_name="s")

    @functools.partial(
        pl.kernel,
        mesh=mesh,
        out_type=jax.ShapeDtypeStruct((B, D), dtype),
        scratch_types=[
            pltpu.VMEM((per_w,), jnp.int32),
            pltpu.VMEM((chunk, D), dtype),
            pltpu.SemaphoreType.DMA,
        ],
    )
    def gk(table_hbm, idx_hbm, out_hbm, idx_v, buf_v, sem):
        wid = lax.axis_index("s") * 2 + lax.axis_index("c")
        pltpu.sync_copy(idx_hbm.at[pl.ds(wid * per_w, per_w)], idx_v)

        def body(c, carry):
            pltpu.async_copy(table_hbm.at[idx_v.at[pl.ds(c * chunk, chunk)]],
                             buf_v, sem).wait()
            pltpu.sync_copy(buf_v, out_hbm.at[pl.ds(wid * per_w + c * chunk, chunk)])
            return carry

        lax.fori_loop(0, n_chunks, body, 0)

    return gk


_gather_feat = _make_sc_gather(N_NODES, D_FEAT, 250880, jnp.float32, 80)
_gather_adj = _make_sc_gather(N_NODES, 128, 2 * BATCH, jnp.int32, 32)
_gather_m1 = _make_sc_gather(N_NODES, HIDDEN, 2 * BATCH * S_HOP1, jnp.float32, 80)
_gather_h10 = _make_sc_gather(N_NODES, 2 * D_FEAT, 2 * BATCH, jnp.float32, 32)

NB = 200


def _fused0_body(g_ref, f_ref, w0_ref, b0_ref, nw0_ref, sw0_ref, w1_ref, b1_ref,
                 h10_ref, m1_ref):
    flat = g_ref[...].reshape(NB * S_HOP2, D_FEAT)
    nh = jnp.dot(flat, w0_ref[...], preferred_element_type=jnp.float32)
    nh3 = nh.reshape(NB, S_HOP2, HIDDEN)
    b0 = b0_ref[...]
    p25 = jax.nn.relu(jnp.max(nh3, axis=1) + b0)
    p10 = jax.nn.relu(jnp.max(nh3[:, :S_HOP1], axis=1) + b0)
    s0 = jnp.dot(f_ref[...], sw0_ref[...], preferred_element_type=jnp.float32)
    n25 = jnp.dot(p25, nw0_ref[...], preferred_element_type=jnp.float32)
    n10 = jnp.dot(p10, nw0_ref[...], preferred_element_type=jnp.float32)
    h25 = jax.nn.relu(jnp.concatenate([s0, n25], axis=1))
    h10_ref[...] = jax.nn.relu(jnp.concatenate([s0, n10], axis=1))
    m1_ref[...] = jax.nn.relu(
        jnp.dot(h25, w1_ref[...], preferred_element_type=jnp.float32) + b1_ref[...])


RB = 256


def _final_body(m_ref, h_ref, sw1_ref, nw1_ref, o_ref):
    pooled = jnp.max(m_ref[...].reshape(RB, S_HOP1, HIDDEN), axis=1)
    a = jnp.dot(h_ref[...], sw1_ref[...], preferred_element_type=jnp.float32)
    b = jnp.dot(pooled, nw1_ref[...], preferred_element_type=jnp.float32)
    o = jnp.concatenate([a, b], axis=1)
    s = jnp.sum(o * o, axis=1, keepdims=True)
    o_ref[...] = o * lax.rsqrt(jnp.maximum(s, 1e-12))


def kernel(batch1, batch2, features, adj, mlp_W0, mlp_b0, neigh_W0, self_W0,
           mlp_W1, mlp_b1, neigh_W1, self_W1):
    idx1 = adj[:, :S_HOP2].reshape(-1)
    idx1 = jnp.concatenate([idx1, jnp.zeros((880,), jnp.int32)])
    g25 = _gather_feat(features, idx1)

    h10_tab, m1_tab = pl.pallas_call(
        _fused0_body,
        grid=(N_NODES // NB,),
        in_specs=[
            pl.BlockSpec((NB * S_HOP2, D_FEAT), lambda i: (i, 0)),
            pl.BlockSpec((NB, D_FEAT), lambda i: (i, 0)),
            pl.BlockSpec((D_FEAT, HIDDEN), lambda i: (0, 0)),
            pl.BlockSpec((1, HIDDEN), lambda i: (0, 0)),
            pl.BlockSpec((HIDDEN, D_FEAT), lambda i: (0, 0)),
            pl.BlockSpec((D_FEAT, D_FEAT), lambda i: (0, 0)),
            pl.BlockSpec((2 * D_FEAT, HIDDEN), lambda i: (0, 0)),
            pl.BlockSpec((1, HIDDEN), lambda i: (0, 0)),
        ],
        out_specs=[
            pl.BlockSpec((NB, 2 * D_FEAT), lambda i: (i, 0)),
            pl.BlockSpec((NB, HIDDEN), lambda i: (i, 0)),
        ],
        out_shape=[
            jax.ShapeDtypeStruct((N_NODES, 2 * D_FEAT), jnp.float32),
            jax.ShapeDtypeStruct((N_NODES, HIDDEN), jnp.float32),
        ],
    )(g25, features, mlp_W0, mlp_b0.reshape(1, HIDDEN),
      neigh_W0, self_W0, mlp_W1, mlp_b1.reshape(1, HIDDEN))

    batch = jnp.concatenate([batch1, batch2])
    adj128 = jnp.pad(adj, ((0, 0), (0, 128 - MAX_DEG)))
    adjb = _gather_adj(adj128, batch)
    idx3 = adjb[:, :S_HOP1].reshape(-1)
    mrows = _gather_m1(m1_tab, idx3)
    h10b = _gather_h10(h10_tab, batch)

    out = pl.pallas_call(
        _final_body,
        grid=(2 * BATCH // RB,),
        in_specs=[
            pl.BlockSpec((RB * S_HOP1, HIDDEN), lambda i: (i, 0)),
            pl.BlockSpec((RB, 2 * D_FEAT), lambda i: (i, 0)),
            pl.BlockSpec((2 * D_FEAT, D_FEAT), lambda i: (0, 0)),
            pl.BlockSpec((HIDDEN, D_FEAT), lambda i: (0, 0)),
        ],
        out_specs=pl.BlockSpec((RB, 2 * D_FEAT), lambda i: (i, 0)),
        out_shape=jax.ShapeDtypeStruct((2 * BATCH, 2 * D_FEAT), jnp.float32),
    )(mrows, h10b, self_W1, neigh_W1)

    return (out[:BATCH], out[BATCH:])
